# Initial kernel scaffold; baseline (speedup 1.0000x reference)
import jax
import jax.numpy as jnp
from jax.experimental import pallas as pl


def kernel(instance_batch_embs, token_embs, instance_edge_index, token_edge_index, instance_batch_token_ids, gat_W1, gat_al1, gat_ar1, gat_W2, gat_al2, gat_ar2, gcn_W1, gcn_W2, Wih0f, Whh0f, b0f, Wih0b, Whh0b, b0b, Wih1f, Whh1f, b1f, Wih1b, Whh1b, b1b, fc_W, fc_b):
    raise NotImplementedError("write your pallas kernel here")


# speedup vs baseline: 1.0000x; 1.0000x over previous
import jax
import jax.numpy as jnp
from jax.experimental import pallas as pl

N_INST = 2048; E_INST = 16384; N_TOK = 10000; E_TOK = 163840
IN_DIM = 256; HID = 256; HEADS = 4; OUT = 100; NCLS = 4; LSTM_H = 200


def _gat_conv(x, src, dst, W, al, ar, N):
    H, D = al.shape
    feat = (x @ W).reshape(-1, H, D)
    el = jnp.sum(feat * al[None, :, :], axis=-1)
    er = jnp.sum(feat * ar[None, :, :], axis=-1)
    e = jax.nn.leaky_relu(el[src] + er[dst], negative_slope=0.2)
    emax = jax.ops.segment_max(e, dst, num_segments=N)
    emax = jnp.where(jnp.isfinite(emax), emax, 0.0)
    ee = jnp.exp(e - emax[dst])
    den = jax.ops.segment_sum(ee, dst, num_segments=N)
    alpha = ee / (den[dst] + 1e-9)
    return jax.ops.segment_sum(feat[src] * alpha[:, :, None], dst, num_segments=N)


def _gcn_layer(x, src, dst, W, N):
    h = x @ W
    deg = jnp.bincount(dst, length=N).astype(jnp.float32) + 1.0
    norm = 1.0 / jnp.sqrt(deg)
    msg = h[src] * (norm[src] * norm[dst])[:, None]
    out = jax.ops.segment_sum(msg, dst, num_segments=N)
    return out + h * (norm * norm)[:, None]


def _lstm_dir(x, Wih, Whh, b, reverse):
    Hh = Whh.shape[0]
    def step(carry, xt):
        h, c = carry
        z = xt @ Wih + h @ Whh + b
        i, f, g, o = jnp.split(z, 4)
        c = jax.nn.sigmoid(f) * c + jax.nn.sigmoid(i) * jnp.tanh(g)
        h = jax.nn.sigmoid(o) * jnp.tanh(c)
        return (h, c), h
    xs = x[::-1] if reverse else x
    (h, c), ys = jax.lax.scan(step, (jnp.zeros((Hh,), x.dtype), jnp.zeros((Hh,), x.dtype)), xs)
    if reverse:
        ys = ys[::-1]
    return ys, h


def _fc_body(h_ref, w_ref, b_ref, o_ref):
    o_ref[...] = jnp.dot(h_ref[...], w_ref[...],
                         preferred_element_type=jnp.float32) + b_ref[...]


def kernel(instance_batch_embs, token_embs, instance_edge_index, token_edge_index, instance_batch_token_ids, gat_W1, gat_al1, gat_ar1, gat_W2, gat_al2, gat_ar2, gcn_W1, gcn_W2, Wih0f, Whh0f, b0f, Wih0b, Whh0b, b0b, Wih1f, Whh1f, b1f, Wih1b, Whh1b, b1b, fc_W, fc_b):
    src_i = instance_edge_index[0]; dst_i = instance_edge_index[1]
    src_t = token_edge_index[0]; dst_t = token_edge_index[1]
    h = jax.nn.relu(_gat_conv(instance_batch_embs, src_i, dst_i, gat_W1, gat_al1, gat_ar1, N_INST))
    h = h.reshape(N_INST, HEADS * HID)
    h = jax.nn.relu(_gat_conv(h, src_i, dst_i, gat_W2, gat_al2, gat_ar2, N_INST))
    inst = h.reshape(N_INST, OUT)
    t = jax.nn.relu(_gcn_layer(token_embs, src_t, dst_t, gcn_W1, N_TOK))
    t = _gcn_layer(t, src_t, dst_t, gcn_W2, N_TOK)
    t = t[instance_batch_token_ids]
    embs = jnp.concatenate([inst, t], axis=0)
    y0f, _h0f = _lstm_dir(embs, Wih0f, Whh0f, b0f, False)
    y0b, _h0b = _lstm_dir(embs, Wih0b, Whh0b, b0b, True)
    x1 = jnp.concatenate([y0f, y0b], axis=1)
    _y1f, h1f = _lstm_dir(x1, Wih1f, Whh1f, b1f, False)
    _y1b, h1b = _lstm_dir(x1, Wih1b, Whh1b, b1b, True)
    hidden = jnp.concatenate([h1f, h1b], axis=0)
    logits = pl.pallas_call(
        _fc_body,
        out_shape=jax.ShapeDtypeStruct((1, NCLS), jnp.float32),
    )(hidden[None, :], fc_W, fc_b[None, :])
    return logits[0]


# speedup vs baseline: 3.7324x; 3.7324x over previous
import jax
import jax.numpy as jnp
from jax.experimental import pallas as pl
from jax.experimental.pallas import tpu as pltpu

N_INST = 2048; E_INST = 16384; N_TOK = 10000; E_TOK = 163840
IN_DIM = 256; HID = 256; HEADS = 4; OUT = 100; NCLS = 4; LSTM_H = 200

SEQ = 2 * N_INST
HP = 256
ZP = 4 * HP
TBLK = 512
GRID = SEQ // TBLK


def _pack_gate_cols(W):
    K = W.shape[0]
    Wp = jnp.zeros((K, ZP), W.dtype)
    for g in range(4):
        Wp = Wp.at[:, g * HP:g * HP + LSTM_H].set(W[:, g * LSTM_H:(g + 1) * LSTM_H])
    return Wp


def _pack_whh(W):
    Wp = jnp.zeros((HP, ZP), W.dtype)
    return Wp.at[:LSTM_H, :].set(_pack_gate_cols(W))


def _pack_rows(W, row0):
    Wp = jnp.zeros((HP, ZP), W.dtype)
    return Wp.at[:LSTM_H, :].set(_pack_gate_cols(W[row0:row0 + LSTM_H]))


def _u0_body(e_ref, wf_ref, wb_ref, bf_ref, bb_ref, uf_ref, ub_ref):
    e = e_ref[...]
    uf_ref[...] = jnp.dot(e, wf_ref[...], preferred_element_type=jnp.float32) + bf_ref[...]
    ub_ref[...] = jnp.dot(e, wb_ref[...], preferred_element_type=jnp.float32) + bb_ref[...]


def _u1_body(yf_ref, yb_ref, wft_ref, wfb_ref, wbt_ref, wbb_ref, bf_ref, bb_ref,
             uf_ref, ub_ref):
    yf = yf_ref[...]
    yb = yb_ref[...]
    uf_ref[...] = (jnp.dot(yf, wft_ref[...], preferred_element_type=jnp.float32)
                   + jnp.dot(yb, wfb_ref[...], preferred_element_type=jnp.float32)
                   + bf_ref[...])
    ub_ref[...] = (jnp.dot(yf, wbt_ref[...], preferred_element_type=jnp.float32)
                   + jnp.dot(yb, wbb_ref[...], preferred_element_type=jnp.float32)
                   + bb_ref[...])


def _cell(z, c):
    i = jax.nn.sigmoid(z[:, 0:HP])
    f = jax.nn.sigmoid(z[:, HP:2 * HP])
    g = jnp.tanh(z[:, 2 * HP:3 * HP])
    o = jax.nn.sigmoid(z[:, 3 * HP:4 * HP])
    c = f * c + i * g
    h = o * jnp.tanh(c)
    return h, c


def _lstm0_body(uf_ref, ub_ref, whf_ref, whb_ref, yf_ref, yb_ref, st_ref):
    g = pl.program_id(0)

    @pl.when(g == 0)
    def _():
        st_ref[...] = jnp.zeros((4, HP), jnp.float32)

    whf = whf_ref[...]
    whb = whb_ref[...]
    init = (st_ref[pl.ds(0, 1), :], st_ref[pl.ds(1, 1), :],
            st_ref[pl.ds(2, 1), :], st_ref[pl.ds(3, 1), :])

    def step(t, carry):
        hf, cf, hb, cb = carry
        zf = uf_ref[pl.ds(t, 1), :] + jnp.dot(hf, whf, preferred_element_type=jnp.float32)
        hf, cf = _cell(zf, cf)
        yf_ref[pl.ds(t, 1), :] = hf
        tb = TBLK - 1 - t
        zb = ub_ref[pl.ds(tb, 1), :] + jnp.dot(hb, whb, preferred_element_type=jnp.float32)
        hb, cb = _cell(zb, cb)
        yb_ref[pl.ds(tb, 1), :] = hb
        return hf, cf, hb, cb

    hf, cf, hb, cb = jax.lax.fori_loop(0, TBLK, step, init)
    st_ref[pl.ds(0, 1), :] = hf
    st_ref[pl.ds(1, 1), :] = cf
    st_ref[pl.ds(2, 1), :] = hb
    st_ref[pl.ds(3, 1), :] = cb


def _lstm1_body(uf_ref, ub_ref, whf_ref, whb_ref, fcw_ref, fcb_ref, out_ref, st_ref):
    g = pl.program_id(0)

    @pl.when(g == 0)
    def _():
        st_ref[...] = jnp.zeros((4, HP), jnp.float32)

    whf = whf_ref[...]
    whb = whb_ref[...]
    init = (st_ref[pl.ds(0, 1), :], st_ref[pl.ds(1, 1), :],
            st_ref[pl.ds(2, 1), :], st_ref[pl.ds(3, 1), :])

    def step(t, carry):
        hf, cf, hb, cb = carry
        zf = uf_ref[pl.ds(t, 1), :] + jnp.dot(hf, whf, preferred_element_type=jnp.float32)
        hf, cf = _cell(zf, cf)
        tb = TBLK - 1 - t
        zb = ub_ref[pl.ds(tb, 1), :] + jnp.dot(hb, whb, preferred_element_type=jnp.float32)
        hb, cb = _cell(zb, cb)
        return hf, cf, hb, cb

    hf, cf, hb, cb = jax.lax.fori_loop(0, TBLK, step, init)
    st_ref[pl.ds(0, 1), :] = hf
    st_ref[pl.ds(1, 1), :] = cf
    st_ref[pl.ds(2, 1), :] = hb
    st_ref[pl.ds(3, 1), :] = cb

    @pl.when(g == GRID - 1)
    def _():
        hid = jnp.concatenate([hf, hb], axis=1)
        out_ref[...] = jnp.dot(hid, fcw_ref[...], preferred_element_type=jnp.float32) + fcb_ref[...]


def _lstm_head(embs, Wih0f, Whh0f, b0f, Wih0b, Whh0b, b0b,
               Wih1f, Whh1f, b1f, Wih1b, Whh1b, b1b, fc_W, fc_b):
    f32 = jnp.float32

    w0f = _pack_gate_cols(Wih0f); w0b = _pack_gate_cols(Wih0b)
    bp0f = _pack_gate_cols(b0f[None, :]); bp0b = _pack_gate_cols(b0b[None, :])
    whh0f = _pack_whh(Whh0f); whh0b = _pack_whh(Whh0b)
    w1ft = _pack_rows(Wih1f, 0); w1fb = _pack_rows(Wih1f, LSTM_H)
    w1bt = _pack_rows(Wih1b, 0); w1bb = _pack_rows(Wih1b, LSTM_H)
    bp1f = _pack_gate_cols(b1f[None, :]); bp1b = _pack_gate_cols(b1b[None, :])
    whh1f = _pack_whh(Whh1f); whh1b = _pack_whh(Whh1b)
    fcw = jnp.zeros((2 * HP, 128), f32)
    fcw = fcw.at[:LSTM_H, :NCLS].set(fc_W[:LSTM_H])
    fcw = fcw.at[HP:HP + LSTM_H, :NCLS].set(fc_W[LSTM_H:])
    fcb = jnp.zeros((1, 128), f32).at[0, :NCLS].set(fc_b)

    din = embs.shape[1]
    full = lambda shp: pl.BlockSpec(shp, lambda g: (0, 0))
    seq_blk = lambda w: pl.BlockSpec((TBLK, w), lambda g: (g, 0))
    rev_blk = lambda w: pl.BlockSpec((TBLK, w), lambda g: (GRID - 1 - g, 0))

    u0f, u0b = pl.pallas_call(
        _u0_body,
        grid=(GRID,),
        in_specs=[seq_blk(din), full((din, ZP)), full((din, ZP)),
                  full((1, ZP)), full((1, ZP))],
        out_specs=[seq_blk(ZP), seq_blk(ZP)],
        out_shape=[jax.ShapeDtypeStruct((SEQ, ZP), f32)] * 2,
    )(embs, w0f, w0b, bp0f, bp0b)

    yf, yb = pl.pallas_call(
        _lstm0_body,
        grid=(GRID,),
        in_specs=[seq_blk(ZP), rev_blk(ZP), full((HP, ZP)), full((HP, ZP))],
        out_specs=[seq_blk(HP), rev_blk(HP)],
        out_shape=[jax.ShapeDtypeStruct((SEQ, HP), f32)] * 2,
        scratch_shapes=[pltpu.VMEM((4, HP), f32)],
    )(u0f, u0b, whh0f, whh0b)

    u1f, u1b = pl.pallas_call(
        _u1_body,
        grid=(GRID,),
        in_specs=[seq_blk(HP), seq_blk(HP), full((HP, ZP)), full((HP, ZP)),
                  full((HP, ZP)), full((HP, ZP)), full((1, ZP)), full((1, ZP))],
        out_specs=[seq_blk(ZP), seq_blk(ZP)],
        out_shape=[jax.ShapeDtypeStruct((SEQ, ZP), f32)] * 2,
    )(yf, yb, w1ft, w1fb, w1bt, w1bb, bp1f, bp1b)

    logits = pl.pallas_call(
        _lstm1_body,
        grid=(GRID,),
        in_specs=[seq_blk(ZP), rev_blk(ZP), full((HP, ZP)), full((HP, ZP)),
                  full((2 * HP, 128)), full((1, 128))],
        out_specs=pl.BlockSpec((1, 128), lambda g: (0, 0)),
        out_shape=jax.ShapeDtypeStruct((1, 128), f32),
        scratch_shapes=[pltpu.VMEM((4, HP), f32)],
    )(u1f, u1b, whh1f, whh1b, fcw, fcb)

    return logits[0, :NCLS]


def _gat_conv(x, src, dst, W, al, ar, N):
    H, D = al.shape
    feat = (x @ W).reshape(-1, H, D)
    el = jnp.sum(feat * al[None, :, :], axis=-1)
    er = jnp.sum(feat * ar[None, :, :], axis=-1)
    e = jax.nn.leaky_relu(el[src] + er[dst], negative_slope=0.2)
    ee = jnp.exp(e)
    den = jax.ops.segment_sum(ee, dst, num_segments=N)
    num = jax.ops.segment_sum(feat[src] * ee[:, :, None], dst, num_segments=N)
    return num / (den + 1e-9)[:, :, None]


def _gcn_layer(x, src, dst, W, N):
    h = x @ W
    deg = jnp.bincount(dst, length=N).astype(jnp.float32) + 1.0
    norm = 1.0 / jnp.sqrt(deg)
    msg = h[src] * (norm[src] * norm[dst])[:, None]
    out = jax.ops.segment_sum(msg, dst, num_segments=N)
    return out + h * (norm * norm)[:, None]


def kernel(instance_batch_embs, token_embs, instance_edge_index, token_edge_index, instance_batch_token_ids, gat_W1, gat_al1, gat_ar1, gat_W2, gat_al2, gat_ar2, gcn_W1, gcn_W2, Wih0f, Whh0f, b0f, Wih0b, Whh0b, b0b, Wih1f, Whh1f, b1f, Wih1b, Whh1b, b1b, fc_W, fc_b):
    src_i = instance_edge_index[0]; dst_i = instance_edge_index[1]
    src_t = token_edge_index[0]; dst_t = token_edge_index[1]
    h = jax.nn.relu(_gat_conv(instance_batch_embs, src_i, dst_i, gat_W1, gat_al1, gat_ar1, N_INST))
    h = h.reshape(N_INST, HEADS * HID)
    h = jax.nn.relu(_gat_conv(h, src_i, dst_i, gat_W2, gat_al2, gat_ar2, N_INST))
    inst = h.reshape(N_INST, OUT)
    t = jax.nn.relu(_gcn_layer(token_embs, src_t, dst_t, gcn_W1, N_TOK))
    t = _gcn_layer(t, src_t, dst_t, gcn_W2, N_TOK)
    t = t[instance_batch_token_ids]
    embs = jnp.concatenate([inst, t], axis=0)
    return _lstm_head(embs, Wih0f, Whh0f, b0f, Wih0b, Whh0b, b0b,
                      Wih1f, Whh1f, b1f, Wih1b, Whh1b, b1b, fc_W, fc_b)


# speedup vs baseline: 6.9738x; 1.8684x over previous
import functools

import jax
import jax.numpy as jnp
from jax import lax
from jax.experimental import pallas as pl
from jax.experimental.pallas import tpu as pltpu
from jax.experimental.pallas import tpu_sc as plsc

N_INST = 2048; E_INST = 16384; N_TOK = 10000; E_TOK = 163840
IN_DIM = 256; HID = 256; HEADS = 4; OUT = 100; NCLS = 4; LSTM_H = 200

SEQ = 2 * N_INST
HP = 256
ZP = 4 * HP
TBLK = 512
GRID = SEQ // TBLK


def _pack_gate_cols(W):
    K = W.shape[0]
    Wp = jnp.zeros((K, ZP), W.dtype)
    for g in range(4):
        Wp = Wp.at[:, g * HP:g * HP + LSTM_H].set(W[:, g * LSTM_H:(g + 1) * LSTM_H])
    return Wp


def _pack_whh(W):
    Wp = jnp.zeros((HP, ZP), W.dtype)
    return Wp.at[:LSTM_H, :].set(_pack_gate_cols(W))


def _pack_rows(W, row0):
    Wp = jnp.zeros((HP, ZP), W.dtype)
    return Wp.at[:LSTM_H, :].set(_pack_gate_cols(W[row0:row0 + LSTM_H]))


def _u0_body(e_ref, wf_ref, wb_ref, bf_ref, bb_ref, uf_ref, ub_ref):
    e = e_ref[...]
    uf_ref[...] = jnp.dot(e, wf_ref[...], preferred_element_type=jnp.float32) + bf_ref[...]
    ub_ref[...] = jnp.dot(e, wb_ref[...], preferred_element_type=jnp.float32) + bb_ref[...]


def _u1_body(yf_ref, yb_ref, wft_ref, wfb_ref, wbt_ref, wbb_ref, bf_ref, bb_ref,
             uf_ref, ub_ref):
    yf = yf_ref[...]
    yb = yb_ref[...]
    uf_ref[...] = (jnp.dot(yf, wft_ref[...], preferred_element_type=jnp.float32)
                   + jnp.dot(yb, wfb_ref[...], preferred_element_type=jnp.float32)
                   + bf_ref[...])
    ub_ref[...] = (jnp.dot(yf, wbt_ref[...], preferred_element_type=jnp.float32)
                   + jnp.dot(yb, wbb_ref[...], preferred_element_type=jnp.float32)
                   + bb_ref[...])


def _cell(z, c):
    i = jax.nn.sigmoid(z[:, 0:HP])
    f = jax.nn.sigmoid(z[:, HP:2 * HP])
    g = jnp.tanh(z[:, 2 * HP:3 * HP])
    o = jax.nn.sigmoid(z[:, 3 * HP:4 * HP])
    c = f * c + i * g
    h = o * jnp.tanh(c)
    return h, c


def _lstm0_body(uf_ref, ub_ref, whf_ref, whb_ref, yf_ref, yb_ref, st_ref):
    g = pl.program_id(0)

    @pl.when(g == 0)
    def _():
        st_ref[...] = jnp.zeros((4, HP), jnp.float32)

    whf = whf_ref[...]
    whb = whb_ref[...]
    init = (st_ref[pl.ds(0, 1), :], st_ref[pl.ds(1, 1), :],
            st_ref[pl.ds(2, 1), :], st_ref[pl.ds(3, 1), :])

    def step(t, carry):
        hf, cf, hb, cb = carry
        zf = uf_ref[pl.ds(t, 1), :] + jnp.dot(hf, whf, preferred_element_type=jnp.float32)
        hf, cf = _cell(zf, cf)
        yf_ref[pl.ds(t, 1), :] = hf
        tb = TBLK - 1 - t
        zb = ub_ref[pl.ds(tb, 1), :] + jnp.dot(hb, whb, preferred_element_type=jnp.float32)
        hb, cb = _cell(zb, cb)
        yb_ref[pl.ds(tb, 1), :] = hb
        return hf, cf, hb, cb

    hf, cf, hb, cb = jax.lax.fori_loop(0, TBLK, step, init)
    st_ref[pl.ds(0, 1), :] = hf
    st_ref[pl.ds(1, 1), :] = cf
    st_ref[pl.ds(2, 1), :] = hb
    st_ref[pl.ds(3, 1), :] = cb


def _lstm1_body(uf_ref, ub_ref, whf_ref, whb_ref, fcw_ref, fcb_ref, out_ref, st_ref):
    g = pl.program_id(0)

    @pl.when(g == 0)
    def _():
        st_ref[...] = jnp.zeros((4, HP), jnp.float32)

    whf = whf_ref[...]
    whb = whb_ref[...]
    init = (st_ref[pl.ds(0, 1), :], st_ref[pl.ds(1, 1), :],
            st_ref[pl.ds(2, 1), :], st_ref[pl.ds(3, 1), :])

    def step(t, carry):
        hf, cf, hb, cb = carry
        zf = uf_ref[pl.ds(t, 1), :] + jnp.dot(hf, whf, preferred_element_type=jnp.float32)
        hf, cf = _cell(zf, cf)
        tb = TBLK - 1 - t
        zb = ub_ref[pl.ds(tb, 1), :] + jnp.dot(hb, whb, preferred_element_type=jnp.float32)
        hb, cb = _cell(zb, cb)
        return hf, cf, hb, cb

    hf, cf, hb, cb = jax.lax.fori_loop(0, TBLK, step, init)
    st_ref[pl.ds(0, 1), :] = hf
    st_ref[pl.ds(1, 1), :] = cf
    st_ref[pl.ds(2, 1), :] = hb
    st_ref[pl.ds(3, 1), :] = cb

    @pl.when(g == GRID - 1)
    def _():
        hid = jnp.concatenate([hf, hb], axis=1)
        out_ref[...] = jnp.dot(hid, fcw_ref[...], preferred_element_type=jnp.float32) + fcb_ref[...]


def _lstm_head(embs, Wih0f, Whh0f, b0f, Wih0b, Whh0b, b0b,
               Wih1f, Whh1f, b1f, Wih1b, Whh1b, b1b, fc_W, fc_b):
    f32 = jnp.float32

    w0f = _pack_gate_cols(Wih0f); w0b = _pack_gate_cols(Wih0b)
    bp0f = _pack_gate_cols(b0f[None, :]); bp0b = _pack_gate_cols(b0b[None, :])
    whh0f = _pack_whh(Whh0f); whh0b = _pack_whh(Whh0b)
    w1ft = _pack_rows(Wih1f, 0); w1fb = _pack_rows(Wih1f, LSTM_H)
    w1bt = _pack_rows(Wih1b, 0); w1bb = _pack_rows(Wih1b, LSTM_H)
    bp1f = _pack_gate_cols(b1f[None, :]); bp1b = _pack_gate_cols(b1b[None, :])
    whh1f = _pack_whh(Whh1f); whh1b = _pack_whh(Whh1b)
    fcw = jnp.zeros((2 * HP, 128), f32)
    fcw = fcw.at[:LSTM_H, :NCLS].set(fc_W[:LSTM_H])
    fcw = fcw.at[HP:HP + LSTM_H, :NCLS].set(fc_W[LSTM_H:])
    fcb = jnp.zeros((1, 128), f32).at[0, :NCLS].set(fc_b)

    din = embs.shape[1]
    full = lambda shp: pl.BlockSpec(shp, lambda g: (0, 0))
    seq_blk = lambda w: pl.BlockSpec((TBLK, w), lambda g: (g, 0))
    rev_blk = lambda w: pl.BlockSpec((TBLK, w), lambda g: (GRID - 1 - g, 0))

    u0f, u0b = pl.pallas_call(
        _u0_body,
        grid=(GRID,),
        in_specs=[seq_blk(din), full((din, ZP)), full((din, ZP)),
                  full((1, ZP)), full((1, ZP))],
        out_specs=[seq_blk(ZP), seq_blk(ZP)],
        out_shape=[jax.ShapeDtypeStruct((SEQ, ZP), f32)] * 2,
    )(embs, w0f, w0b, bp0f, bp0b)

    yf, yb = pl.pallas_call(
        _lstm0_body,
        grid=(GRID,),
        in_specs=[seq_blk(ZP), rev_blk(ZP), full((HP, ZP)), full((HP, ZP))],
        out_specs=[seq_blk(HP), rev_blk(HP)],
        out_shape=[jax.ShapeDtypeStruct((SEQ, HP), f32)] * 2,
        scratch_shapes=[pltpu.VMEM((4, HP), f32)],
    )(u0f, u0b, whh0f, whh0b)

    u1f, u1b = pl.pallas_call(
        _u1_body,
        grid=(GRID,),
        in_specs=[seq_blk(HP), seq_blk(HP), full((HP, ZP)), full((HP, ZP)),
                  full((HP, ZP)), full((HP, ZP)), full((1, ZP)), full((1, ZP))],
        out_specs=[seq_blk(ZP), seq_blk(ZP)],
        out_shape=[jax.ShapeDtypeStruct((SEQ, ZP), f32)] * 2,
    )(yf, yb, w1ft, w1fb, w1bt, w1bb, bp1f, bp1b)

    logits = pl.pallas_call(
        _lstm1_body,
        grid=(GRID,),
        in_specs=[seq_blk(ZP), rev_blk(ZP), full((HP, ZP)), full((HP, ZP)),
                  full((2 * HP, 128)), full((1, 128))],
        out_specs=pl.BlockSpec((1, 128), lambda g: (0, 0)),
        out_shape=jax.ShapeDtypeStruct((1, 128), f32),
        scratch_shapes=[pltpu.VMEM((4, HP), f32)],
    )(u1f, u1b, whh1f, whh1b, fcw, fcb)

    return logits[0, :NCLS]


_SC_MESH = dict(core_axis_name="c", subcore_axis_name="s")
NSC = 2
NTILE = 16
NW = NSC * NTILE


def _wid():
    return lax.axis_index("s") * NSC + lax.axis_index("c")


def _sc_gather_body(rows_per_w, ncols, table_hbm, ids_hbm, out_hbm, idx_v, rows_v, sem):
    base = _wid() * rows_per_w
    pltpu.sync_copy(ids_hbm.at[pl.ds(base, rows_per_w)], idx_v)
    pltpu.async_copy(table_hbm.at[idx_v], rows_v, sem).wait()
    pltpu.sync_copy(rows_v, out_hbm.at[pl.ds(base, rows_per_w)])


def _sc_gather(table, ids):
    B = ids.shape[0]
    C = table.shape[1]
    rpw = B // NW
    f = pl.kernel(
        functools.partial(_sc_gather_body, rpw, C),
        mesh=plsc.VectorSubcoreMesh(**_SC_MESH),
        out_type=jax.ShapeDtypeStruct((B, C), jnp.float32),
        scratch_types=[
            pltpu.VMEM((rpw,), jnp.int32),
            pltpu.VMEM((rpw, C), jnp.float32),
            pltpu.SemaphoreType.DMA,
        ],
    )
    return f(table, ids)


NTP = 10240
CHUNK = 128


def _deg_body(dst_hbm, ones_hbm, zeros_hbm, out_hbm, dst_v, ones_v, acc_sh, sem):
    c = lax.axis_index("c")
    s = lax.axis_index("s")
    per_t = E_TOK // NW
    rows_t = NTP // NTILE
    pltpu.sync_copy(zeros_hbm, acc_sh.at[pl.ds(s * rows_t, rows_t)])
    pltpu.sync_copy(ones_hbm, ones_v)
    plsc.subcore_barrier()

    def chunk(ch, _):
        base = (c * NTILE + s) * per_t + ch * CHUNK
        pltpu.sync_copy(dst_hbm.at[pl.ds(base, CHUNK)], dst_v)
        pltpu.sync_copy(ones_v, acc_sh.at[dst_v], add=True)
        return 0
    lax.fori_loop(0, per_t // CHUNK, chunk, 0)

    plsc.subcore_barrier()
    pltpu.sync_copy(acc_sh.at[pl.ds(s * rows_t, rows_t)],
                    out_hbm.at[pl.ds(c * NTP + s * rows_t, rows_t)])


def _sc_degree(dst):
    f = pl.kernel(
        _deg_body,
        mesh=plsc.VectorSubcoreMesh(**_SC_MESH),
        out_type=jax.ShapeDtypeStruct((2 * NTP, 128), jnp.float32),
        scratch_types=[
            pltpu.VMEM((CHUNK,), jnp.int32),
            pltpu.VMEM((CHUNK, 128), jnp.float32),
            pltpu.VMEM_SHARED((NTP, 128), jnp.float32),
            pltpu.SemaphoreType.DMA,
        ],
    )
    ones = jnp.ones((CHUNK, 128), jnp.float32)
    zeros = jnp.zeros((NTP // NTILE, 128), jnp.float32)
    return f(dst, ones, zeros)


def _segsum_fsplit_body(n_edges, n_rows, src_hbm, dst_hbm, table_hbm, zeros_hbm,
                        out_hbm, src_v, dst_v, rows_v, acc_sh, sem):
    c = lax.axis_index("c")
    s = lax.axis_index("s")
    per_t = n_edges // NTILE
    rows_t = n_rows // NTILE
    for i in range(rows_t // CHUNK):
        pltpu.sync_copy(zeros_hbm, acc_sh.at[pl.ds(s * rows_t + i * CHUNK, CHUNK)])
    plsc.subcore_barrier()

    def chunk(ch, _):
        base = s * per_t + ch * CHUNK
        pltpu.sync_copy(src_hbm.at[pl.ds(base, CHUNK)], src_v)
        pltpu.sync_copy(dst_hbm.at[pl.ds(base, CHUNK)], dst_v)
        off = c * n_rows
        for k in range(CHUNK // 16):
            src_v[pl.ds(k * 16, 16)] = src_v[pl.ds(k * 16, 16)] + off
        pltpu.async_copy(table_hbm.at[src_v], rows_v, sem).wait()
        pltpu.sync_copy(rows_v, acc_sh.at[dst_v], add=True)
        return 0
    lax.fori_loop(0, per_t // CHUNK, chunk, 0)

    plsc.subcore_barrier()
    pltpu.sync_copy(acc_sh.at[pl.ds(s * rows_t, rows_t)],
                    out_hbm.at[pl.ds(c * n_rows + s * rows_t, rows_t)])


def _sc_segsum_fsplit(src, dst, table_cat, n_edges, n_rows):
    f = pl.kernel(
        functools.partial(_segsum_fsplit_body, n_edges, n_rows),
        mesh=plsc.VectorSubcoreMesh(**_SC_MESH),
        out_type=jax.ShapeDtypeStruct((2 * n_rows, 128), jnp.float32),
        scratch_types=[
            pltpu.VMEM((CHUNK,), jnp.int32),
            pltpu.VMEM((CHUNK,), jnp.int32),
            pltpu.VMEM((CHUNK, 128), jnp.float32),
            pltpu.VMEM_SHARED((n_rows, 128), jnp.float32),
            pltpu.SemaphoreType.DMA,
        ],
    )
    zeros = jnp.zeros((CHUNK, 128), jnp.float32)
    return f(src, dst, table_cat, zeros)


def _segsum_esplit_body(n_edges, n_rows, src_hbm, dst_hbm, table_hbm, zeros_hbm,
                        out_hbm, src_v, dst_v, rows_v, acc_sh, sem):
    c = lax.axis_index("c")
    s = lax.axis_index("s")
    per_t = n_edges // NW
    rows_t = n_rows // NTILE
    for i in range(rows_t // CHUNK):
        pltpu.sync_copy(zeros_hbm, acc_sh.at[pl.ds(s * rows_t + i * CHUNK, CHUNK)])
    plsc.subcore_barrier()

    def chunk(ch, _):
        base = (c * NTILE + s) * per_t + ch * CHUNK
        pltpu.sync_copy(src_hbm.at[pl.ds(base, CHUNK)], src_v)
        pltpu.sync_copy(dst_hbm.at[pl.ds(base, CHUNK)], dst_v)
        pltpu.async_copy(table_hbm.at[src_v], rows_v, sem).wait()
        pltpu.sync_copy(rows_v, acc_sh.at[dst_v], add=True)
        return 0
    lax.fori_loop(0, per_t // CHUNK, chunk, 0)

    plsc.subcore_barrier()
    pltpu.sync_copy(acc_sh.at[pl.ds(s * rows_t, rows_t)],
                    out_hbm.at[pl.ds(c * n_rows + s * rows_t, rows_t)])


def _sc_segsum_esplit(src, dst, table, n_edges, n_rows):
    f = pl.kernel(
        functools.partial(_segsum_esplit_body, n_edges, n_rows),
        mesh=plsc.VectorSubcoreMesh(**_SC_MESH),
        out_type=jax.ShapeDtypeStruct((2 * n_rows, 128), jnp.float32),
        scratch_types=[
            pltpu.VMEM((CHUNK,), jnp.int32),
            pltpu.VMEM((CHUNK,), jnp.int32),
            pltpu.VMEM((CHUNK, 128), jnp.float32),
            pltpu.VMEM_SHARED((n_rows, 128), jnp.float32),
            pltpu.SemaphoreType.DMA,
        ],
    )
    zeros = jnp.zeros((CHUNK, 128), jnp.float32)
    return f(src, dst, table, zeros)


def _m5_body(pa_ref, pb_ref, o_ref):
    o_ref[...] = lax.rsqrt(pa_ref[:, :1] + pb_ref[:, :1] + 1.0)


def _m4_body(x_ref, w_ref, nc_ref, o_ref):
    o_ref[...] = jnp.dot(x_ref[...], w_ref[...],
                         preferred_element_type=jnp.float32) * nc_ref[...]


def _m6_body(sc_ref, hs_ref, nc_ref, o_ref):
    nc = nc_ref[...]
    o_ref[...] = jax.nn.relu(nc * sc_ref[...] + hs_ref[...] * nc)


def _m7_body(ta_ref, tb_ref, wa_ref, wb_ref, nc_ref, o_ref):
    o_ref[...] = (jnp.dot(ta_ref[...], wa_ref[...], preferred_element_type=jnp.float32)
                  + jnp.dot(tb_ref[...], wb_ref[...], preferred_element_type=jnp.float32)
                  ) * nc_ref[...]


def _m8_body(p0_ref, p1_ref, hs_ref, nc_ref, o_ref):
    o_ref[...] = nc_ref[...] * (p0_ref[...] + p1_ref[...] + hs_ref[...])


def _gcn_pipeline(token_embs, src_t, dst_t, gcn_W1, gcn_W2, ids):
    f32 = jnp.float32
    RB = 1024
    GR = NTP // RB
    tokp = jnp.pad(token_embs, ((0, NTP - N_TOK), (0, 0)))

    deg_parts = _sc_degree(dst_t)
    norm_col = pl.pallas_call(
        _m5_body,
        grid=(GR,),
        in_specs=[pl.BlockSpec((RB, 128), lambda r: (r, 0)),
                  pl.BlockSpec((RB, 128), lambda r: (r + GR, 0))],
        out_specs=pl.BlockSpec((RB, 1), lambda r: (r, 0)),
        out_shape=jax.ShapeDtypeStruct((NTP, 1), f32),
    )(deg_parts, deg_parts)

    h1cat = pl.pallas_call(
        _m4_body,
        grid=(GR, 2),
        in_specs=[pl.BlockSpec((RB, IN_DIM), lambda r, c: (r, 0)),
                  pl.BlockSpec((IN_DIM, 128), lambda r, c: (0, c)),
                  pl.BlockSpec((RB, 1), lambda r, c: (r, 0))],
        out_specs=pl.BlockSpec((RB, 128), lambda r, c: (r + GR * c, 0)),
        out_shape=jax.ShapeDtypeStruct((2 * NTP, 128), f32),
    )(tokp, gcn_W1, norm_col)

    scat1 = _sc_segsum_fsplit(src_t, dst_t, h1cat, E_TOK, NTP)

    t1cat = pl.pallas_call(
        _m6_body,
        grid=(GR, 2),
        in_specs=[pl.BlockSpec((RB, 128), lambda r, c: (r + GR * c, 0)),
                  pl.BlockSpec((RB, 128), lambda r, c: (r + GR * c, 0)),
                  pl.BlockSpec((RB, 1), lambda r, c: (r, 0))],
        out_specs=pl.BlockSpec((RB, 128), lambda r, c: (r + GR * c, 0)),
        out_shape=jax.ShapeDtypeStruct((2 * NTP, 128), f32),
    )(scat1, h1cat, norm_col)

    w2a = jnp.zeros((128, 128), f32).at[:, :OUT].set(gcn_W2[:128])
    w2b = jnp.zeros((128, 128), f32).at[:, :OUT].set(gcn_W2[128:])
    h2s = pl.pallas_call(
        _m7_body,
        grid=(GR,),
        in_specs=[pl.BlockSpec((RB, 128), lambda r: (r, 0)),
                  pl.BlockSpec((RB, 128), lambda r: (r + GR, 0)),
                  pl.BlockSpec((128, 128), lambda r: (0, 0)),
                  pl.BlockSpec((128, 128), lambda r: (0, 0)),
                  pl.BlockSpec((RB, 1), lambda r: (r, 0))],
        out_specs=pl.BlockSpec((RB, 128), lambda r: (r, 0)),
        out_shape=jax.ShapeDtypeStruct((NTP, 128), f32),
    )(t1cat, t1cat, w2a, w2b, norm_col)

    scat2 = _sc_segsum_esplit(src_t, dst_t, h2s, E_TOK, NTP)

    t2 = pl.pallas_call(
        _m8_body,
        grid=(GR,),
        in_specs=[pl.BlockSpec((RB, 128), lambda r: (r, 0)),
                  pl.BlockSpec((RB, 128), lambda r: (r + GR, 0)),
                  pl.BlockSpec((RB, 128), lambda r: (r, 0)),
                  pl.BlockSpec((RB, 1), lambda r: (r, 0))],
        out_specs=pl.BlockSpec((RB, 128), lambda r: (r, 0)),
        out_shape=jax.ShapeDtypeStruct((NTP, 128), f32),
    )(scat2, scat2, h2s, norm_col)

    return _sc_gather(t2, ids)


def _gat_conv(x, src, dst, W, al, ar, N):
    H, D = al.shape
    feat = (x @ W).reshape(-1, H, D)
    el = jnp.sum(feat * al[None, :, :], axis=-1)
    er = jnp.sum(feat * ar[None, :, :], axis=-1)
    e = jax.nn.leaky_relu(el[src] + er[dst], negative_slope=0.2)
    ee = jnp.exp(e)
    den = jax.ops.segment_sum(ee, dst, num_segments=N)
    num = jax.ops.segment_sum(feat[src] * ee[:, :, None], dst, num_segments=N)
    return num / (den + 1e-9)[:, :, None]


def _gcn_layer(x, src, dst, W, N):
    h = x @ W
    deg = jnp.bincount(dst, length=N).astype(jnp.float32) + 1.0
    norm = 1.0 / jnp.sqrt(deg)
    msg = h[src] * (norm[src] * norm[dst])[:, None]
    out = jax.ops.segment_sum(msg, dst, num_segments=N)
    return out + h * (norm * norm)[:, None]


def kernel(instance_batch_embs, token_embs, instance_edge_index, token_edge_index, instance_batch_token_ids, gat_W1, gat_al1, gat_ar1, gat_W2, gat_al2, gat_ar2, gcn_W1, gcn_W2, Wih0f, Whh0f, b0f, Wih0b, Whh0b, b0b, Wih1f, Whh1f, b1f, Wih1b, Whh1b, b1b, fc_W, fc_b):
    src_i = instance_edge_index[0]; dst_i = instance_edge_index[1]
    src_t = token_edge_index[0]; dst_t = token_edge_index[1]
    h = jax.nn.relu(_gat_conv(instance_batch_embs, src_i, dst_i, gat_W1, gat_al1, gat_ar1, N_INST))
    h = h.reshape(N_INST, HEADS * HID)
    h = jax.nn.relu(_gat_conv(h, src_i, dst_i, gat_W2, gat_al2, gat_ar2, N_INST))
    inst = h.reshape(N_INST, OUT)
    t = _gcn_pipeline(token_embs, src_t, dst_t, gcn_W1, gcn_W2,
                      instance_batch_token_ids)[:, :OUT]
    embs = jnp.concatenate([inst, t], axis=0)
    return _lstm_head(embs, Wih0f, Whh0f, b0f, Wih0b, Whh0b, b0b,
                      Wih1f, Whh1f, b1f, Wih1b, Whh1b, b1b, fc_W, fc_b)
